# baseline (speedup 1.0000x reference)
import functools

import jax
import jax.numpy as jnp
from jax import lax
from jax.experimental import pallas as pl
from jax.experimental.pallas import tpu as pltpu
from jax.experimental.pallas import tpu_sc as plsc

N_NODES = 10000
D = 128
N_EDGES = 320000

NC, NS = 2, 16
NW = NC * NS
EPT = N_EDGES // NW
CHUNK = 64
NFULL = EPT // CHUNK
REM = EPT - NFULL * CHUNK
ROWS_BLK = 1024
NBLK = 10
N_PAD = ROWS_BLK * NBLK
ROWS_PER_TILE = N_PAD // NS
ZROWS = 128

_mesh = plsc.VectorSubcoreMesh(core_axis_name="c", subcore_axis_name="s")


def _sc_deg_body(dst_hbm, z1_hbm, degp_hbm, dst_v, deg_v):
    c = lax.axis_index("c")
    s = lax.axis_index("s")
    wid = c * NS + s
    one16 = jnp.ones((16,), jnp.float32)

    pltpu.sync_copy(z1_hbm, deg_v)
    pltpu.sync_copy(dst_hbm.at[pl.ds(wid * EPT, EPT)], dst_v)

    def scat_body(i, _):
        for u in range(5):
            idx = dst_v[pl.ds((i * 5 + u) * 16, 16)]
            plsc.addupdate_scatter(deg_v, [idx], one16)
        return 0

    lax.fori_loop(0, EPT // 80, scat_body, 0)
    for j in range(NBLK):
        pltpu.sync_copy(deg_v.at[pl.ds(j * ROWS_BLK, ROWS_BLK)],
                        degp_hbm.at[j, wid])


@functools.partial(
    pl.kernel,
    out_type=jax.ShapeDtypeStruct((NBLK, NW, ROWS_BLK), jnp.float32),
    mesh=_mesh,
    scratch_types=[
        pltpu.VMEM((EPT,), jnp.int32),
        pltpu.VMEM((N_PAD,), jnp.float32),
    ],
    compiler_params=pltpu.CompilerParams(needs_layout_passes=False),
)
def _sc_deg(dst_hbm, z1_hbm, degp_hbm, dst_v, deg_v):
    _sc_deg_body(dst_hbm, z1_hbm, degp_hbm, dst_v, deg_v)


def _sc_scatter_body(y_hbm, src_hbm, dst_hbm, z2_hbm, accp_hbm,
                     src_v0, src_v1, dst_v0, dst_v1, dst_v2, dst_v3, dst16_v,
                     rows0, rows1, rows2, acc_sh, gsem, ssem, isem):
    c = lax.axis_index("c")
    s = lax.axis_index("s")
    wid = c * NS + s
    base = wid * EPT

    pltpu.async_copy(z2_hbm, acc_sh.at[pl.ds(s * ROWS_PER_TILE, ROWS_PER_TILE)], ssem)
    pltpu.async_copy(src_hbm.at[pl.ds(base, CHUNK)], src_v0, isem)
    pltpu.async_copy(dst_hbm.at[pl.ds(base, CHUNK)], dst_v0, isem)
    pltpu.make_async_copy(src_hbm.at[pl.ds(base, CHUNK)], src_v0, isem).wait()
    pltpu.async_copy(y_hbm.at[src_v0], rows0, gsem)
    pltpu.make_async_copy(dst_hbm.at[pl.ds(base, CHUNK)], dst_v0, isem).wait()
    pltpu.make_async_copy(
        z2_hbm, acc_sh.at[pl.ds(s * ROWS_PER_TILE, ROWS_PER_TILE)], ssem).wait()
    plsc.subcore_barrier()
    src_ring = (src_v0, src_v1)
    dst_ring = (dst_v0, dst_v1, dst_v2, dst_v3)
    rows_ring = (rows0, rows1, rows2)
    UN = 12
    NJ = NFULL // UN

    pltpu.async_copy(src_hbm.at[pl.ds(base + CHUNK, CHUNK)], src_v1, isem)
    pltpu.async_copy(dst_hbm.at[pl.ds(base + CHUNK, CHUNK)], dst_v1, isem)

    def body(j, _):
        for k in range(UN):
            off = base + (j * UN + k) * CHUNK
            sv, svn = src_ring[k % 2], src_ring[(k + 1) % 2]
            dv, dvn, dv2 = dst_ring[k % 4], dst_ring[(k + 1) % 4], dst_ring[(k + 2) % 4]
            rv, rvn = rows_ring[k % 3], rows_ring[(k + 1) % 3]
            pvr, pvd = rows_ring[(k + 1) % 3], dst_ring[(k + 2) % 4]

            if k == UN - 1:
                @pl.when(j < NJ - 1)
                def _():
                    pltpu.make_async_copy(src_hbm.at[pl.ds(off + CHUNK, CHUNK)], svn, isem).wait()
                    pltpu.make_async_copy(dst_hbm.at[pl.ds(off + CHUNK, CHUNK)], dvn, isem).wait()
            else:
                pltpu.make_async_copy(src_hbm.at[pl.ds(off + CHUNK, CHUNK)], svn, isem).wait()
                pltpu.make_async_copy(dst_hbm.at[pl.ds(off + CHUNK, CHUNK)], dvn, isem).wait()

            pltpu.make_async_copy(y_hbm.at[sv], rv, gsem).wait()

            if k < 2:
                @pl.when(j > 0)
                def _():
                    pltpu.make_async_copy(pvr, acc_sh.at[pvd], ssem).wait()
            else:
                pltpu.make_async_copy(pvr, acc_sh.at[pvd], ssem).wait()

            if k == UN - 1:
                @pl.when(j < NJ - 1)
                def _():
                    pltpu.async_copy(y_hbm.at[svn], rvn, gsem)
            else:
                pltpu.async_copy(y_hbm.at[svn], rvn, gsem)

            pltpu.async_copy(rv, acc_sh.at[dv], ssem, add=True)

            if k >= UN - 2:
                @pl.when(j < NJ - 1)
                def _():
                    pltpu.async_copy(src_hbm.at[pl.ds(off + 2 * CHUNK, CHUNK)], sv, isem)
                    pltpu.async_copy(dst_hbm.at[pl.ds(off + 2 * CHUNK, CHUNK)], dv2, isem)
            else:
                pltpu.async_copy(src_hbm.at[pl.ds(off + 2 * CHUNK, CHUNK)], sv, isem)
                pltpu.async_copy(dst_hbm.at[pl.ds(off + 2 * CHUNK, CHUNK)], dv2, isem)
        return 0

    lax.fori_loop(0, NJ, body, 0)

    pltpu.make_async_copy(rows1, acc_sh.at[dst_v2], ssem).wait()
    pltpu.make_async_copy(rows2, acc_sh.at[dst_v3], ssem).wait()

    off = base + NFULL * CHUNK
    pltpu.sync_copy(src_hbm.at[pl.ds(off, REM)], src_v0.at[pl.ds(0, REM)])
    pltpu.async_copy(y_hbm.at[src_v0.at[pl.ds(0, REM)]],
                     rows0.at[pl.ds(0, REM)], gsem).wait()
    pltpu.sync_copy(dst_hbm.at[pl.ds(off, REM)], dst16_v)
    pltpu.sync_copy(rows0.at[pl.ds(0, REM)], acc_sh.at[dst16_v], add=True)

    plsc.subcore_barrier()
    pltpu.sync_copy(acc_sh.at[pl.ds(s * ROWS_PER_TILE, ROWS_PER_TILE)],
                    accp_hbm.at[c, pl.ds(s * ROWS_PER_TILE, ROWS_PER_TILE)])


@functools.partial(
    pl.kernel,
    out_type=jax.ShapeDtypeStruct((NC, N_PAD, D), jnp.float32),
    mesh=_mesh,
    scratch_types=(
        [pltpu.VMEM((CHUNK,), jnp.int32)] * 6
        + [pltpu.VMEM((REM,), jnp.int32)]
        + [pltpu.VMEM((CHUNK, D), jnp.float32)] * 3
        + [pltpu.VMEM_SHARED((N_PAD, D), jnp.float32)]
        + [pltpu.SemaphoreType.DMA] * 3
    ),
    compiler_params=pltpu.CompilerParams(needs_layout_passes=False),
)
def _sc_scatter(y_hbm, src_hbm, dst_hbm, z2_hbm, accp_hbm,
                src_v0, src_v1, dst_v0, dst_v1, dst_v2, dst_v3, dst16_v,
                rows0, rows1, rows2, acc_sh, gsem, ssem, isem):
    _sc_scatter_body(y_hbm, src_hbm, dst_hbm, z2_hbm, accp_hbm,
                     src_v0, src_v1, dst_v0, dst_v1, dst_v2, dst_v3, dst16_v,
                     rows0, rows1, rows2, acc_sh, gsem, ssem, isem)


def _dinv_bcast(degp_ref):
    deg = 1.0 + jnp.sum(degp_ref[0], axis=0, keepdims=True)
    dinv = lax.rsqrt(deg)
    return lax.dot_general(dinv, jnp.ones((1, D), jnp.float32),
                           (((0,), (0,)), ((), ())),
                           preferred_element_type=jnp.float32)


def _tc_a_body(x_ref, w_ref, degp_ref, xw_ref, y_ref):
    db = _dinv_bcast(degp_ref)
    xw = jnp.dot(x_ref[...], w_ref[...], preferred_element_type=jnp.float32)
    xw_ref[...] = xw
    y_ref[...] = db * xw


def _tc_a(x, W1, degp):
    return pl.pallas_call(
        _tc_a_body,
        grid=(NBLK,),
        in_specs=[
            pl.BlockSpec((ROWS_BLK, D), lambda i: (i, 0)),
            pl.BlockSpec((D, D), lambda i: (0, 0)),
            pl.BlockSpec((1, NW, ROWS_BLK), lambda i: (i, 0, 0)),
        ],
        out_specs=[pl.BlockSpec((ROWS_BLK, D), lambda i: (i, 0))] * 2,
        out_shape=[jax.ShapeDtypeStruct((N_NODES, D), jnp.float32)] * 2,
    )(x, W1, degp)


def _tc_mid_body(accp_ref, degp_ref, xw1_ref, b_ref, a_ref, w_ref,
                 xw2_ref, y2_ref):
    acc = accp_ref[0] + accp_ref[1]
    db = _dinv_bcast(degp_ref)
    z = db * acc + db * db * xw1_ref[...] + b_ref[...]
    h = jnp.where(z >= 0, z, a_ref[...] * z)
    xw2 = jnp.dot(h, w_ref[...], preferred_element_type=jnp.float32)
    xw2_ref[...] = xw2
    y2_ref[...] = db * xw2


def _tc_mid(accp, degp, xw1, b1, a_b, W2):
    return pl.pallas_call(
        _tc_mid_body,
        grid=(NBLK,),
        in_specs=[
            pl.BlockSpec((NC, ROWS_BLK, D), lambda i: (0, i, 0)),
            pl.BlockSpec((1, NW, ROWS_BLK), lambda i: (i, 0, 0)),
            pl.BlockSpec((ROWS_BLK, D), lambda i: (i, 0)),
            pl.BlockSpec((1, D), lambda i: (0, 0)),
            pl.BlockSpec((1, D), lambda i: (0, 0)),
            pl.BlockSpec((D, D), lambda i: (0, 0)),
        ],
        out_specs=[pl.BlockSpec((ROWS_BLK, D), lambda i: (i, 0))] * 2,
        out_shape=[jax.ShapeDtypeStruct((N_NODES, D), jnp.float32)] * 2,
    )(accp, degp, xw1, b1, a_b, W2)


def _tc_out_body(accp_ref, degp_ref, xw2_ref, b_ref, a_ref, out_ref):
    acc = accp_ref[0] + accp_ref[1]
    db = _dinv_bcast(degp_ref)
    z = db * acc + db * db * xw2_ref[...] + b_ref[...]
    out_ref[...] = jnp.where(z >= 0, z, a_ref[...] * z)


def _tc_out(accp, degp, xw2, b2, a_b):
    return pl.pallas_call(
        _tc_out_body,
        grid=(NBLK,),
        in_specs=[
            pl.BlockSpec((NC, ROWS_BLK, D), lambda i: (0, i, 0)),
            pl.BlockSpec((1, NW, ROWS_BLK), lambda i: (i, 0, 0)),
            pl.BlockSpec((ROWS_BLK, D), lambda i: (i, 0)),
            pl.BlockSpec((1, D), lambda i: (0, 0)),
            pl.BlockSpec((1, D), lambda i: (0, 0)),
        ],
        out_specs=pl.BlockSpec((ROWS_BLK, D), lambda i: (i, 0)),
        out_shape=jax.ShapeDtypeStruct((N_NODES, D), jnp.float32),
    )(accp, degp, xw2, b2, a_b)


def kernel(x, edge_index, W1, b1, W2, b2, prelu_a):
    src = edge_index[0]
    dst = edge_index[1]
    b1r = jnp.reshape(b1, (1, D))
    b2r = jnp.reshape(b2, (1, D))
    a_b = jnp.broadcast_to(jnp.reshape(prelu_a, (1, 1)), (1, D))

    z1 = jnp.zeros((N_PAD,), jnp.float32)
    z2 = jnp.zeros((ROWS_PER_TILE, D), jnp.float32)

    degp = _sc_deg(dst, z1)
    xw1, y1 = _tc_a(x, W1, degp)
    accp1 = _sc_scatter(y1, src, dst, z2)
    xw2, y2 = _tc_mid(accp1, degp, xw1, b1r, a_b, W2)
    accp2 = _sc_scatter(y2, src, dst, z2)
    return _tc_out(accp2, degp, xw2, b2r, a_b)


# speedup vs baseline: 1.2622x; 1.2622x over previous
import functools

import jax
import jax.numpy as jnp
from jax import lax
from jax.experimental import pallas as pl
from jax.experimental.pallas import tpu as pltpu
from jax.experimental.pallas import tpu_sc as plsc

N_NODES = 10000
D = 128
N_EDGES = 320000

NC, NS = 2, 16
NW = NC * NS
EPT = N_EDGES // NW
CHUNK = 128
TROWS = 80
EROWS = TROWS * NW
GRP = 8
NGRP = TROWS // GRP
ROWS_BLK = 1024
NBLK = 10
N_PAD = ROWS_BLK * NBLK
ROWS_PER_TILE = N_PAD // NS
ZROWS = 128

_mesh = plsc.VectorSubcoreMesh(core_axis_name="c", subcore_axis_name="s")


def _sc_deg_body(dst_hbm, z1_hbm, degp_hbm, dst_v, deg_v):
    c = lax.axis_index("c")
    s = lax.axis_index("s")
    wid = c * NS + s
    one16 = jnp.ones((16,), jnp.float32)

    pltpu.sync_copy(z1_hbm, deg_v)
    pltpu.sync_copy(dst_hbm.at[pl.ds(wid * EPT, EPT)], dst_v)

    def scat_body(i, _):
        for u in range(5):
            idx = dst_v[pl.ds((i * 5 + u) * 16, 16)]
            plsc.addupdate_scatter(deg_v, [idx], one16)
        return 0

    lax.fori_loop(0, EPT // 80, scat_body, 0)
    for j in range(NBLK):
        pltpu.sync_copy(deg_v.at[pl.ds(j * ROWS_BLK, ROWS_BLK)],
                        degp_hbm.at[j, wid])


@functools.partial(
    pl.kernel,
    out_type=jax.ShapeDtypeStruct((NBLK, NW, ROWS_BLK), jnp.float32),
    mesh=_mesh,
    scratch_types=[
        pltpu.VMEM((EPT,), jnp.int32),
        pltpu.VMEM((N_PAD,), jnp.float32),
    ],
    compiler_params=pltpu.CompilerParams(needs_layout_passes=False),
)
def _sc_deg(dst_hbm, z1_hbm, degp_hbm, dst_v, deg_v):
    _sc_deg_body(dst_hbm, z1_hbm, degp_hbm, dst_v, deg_v)


def _sc_scatter_body(y_hbm, src2_hbm, dst2_hbm, z2_hbm, accp_hbm,
                     srcgA, srcgB, dstgA, dstgB,
                     rows0, rows1, acc_sh, gsem, ssem, isem):
    c = lax.axis_index("c")
    s = lax.axis_index("s")
    wid = c * NS + s
    base_row = wid * TROWS
    rows_ring = (rows0, rows1)
    sbuf = (srcgA, srcgB)
    dbuf = (dstgA, dstgB)

    def _grp_load(g, sb, db):
        pltpu.async_copy(src2_hbm.at[pl.ds(base_row + g * GRP, GRP)], sb, isem)
        pltpu.async_copy(dst2_hbm.at[pl.ds(base_row + g * GRP, GRP)], db, isem)

    def _grp_wait(g, sb, db):
        pltpu.make_async_copy(src2_hbm.at[pl.ds(base_row + g * GRP, GRP)], sb, isem).wait()
        pltpu.make_async_copy(dst2_hbm.at[pl.ds(base_row + g * GRP, GRP)], db, isem).wait()

    pltpu.async_copy(z2_hbm, acc_sh.at[pl.ds(s * ROWS_PER_TILE, ROWS_PER_TILE)], ssem)
    _grp_load(0, srcgA, dstgA)
    _grp_load(1, srcgB, dstgB)
    _grp_wait(0, srcgA, dstgA)
    pltpu.async_copy(y_hbm.at[srcgA.at[0]], rows0, gsem)
    pltpu.make_async_copy(
        z2_hbm, acc_sh.at[pl.ds(s * ROWS_PER_TILE, ROWS_PER_TILE)], ssem).wait()
    plsc.subcore_barrier()

    def body(g2, _):
        for gg in range(2):
            g = g2 * 2 + gg
            bS, bD = sbuf[gg], dbuf[gg]
            oS, oD = sbuf[1 - gg], dbuf[1 - gg]
            for k in range(GRP):
                rv, rvn = rows_ring[k % 2], rows_ring[(k + 1) % 2]
                pltpu.make_async_copy(y_hbm.at[bS.at[k]], rv, gsem).wait()
                if k > 0:
                    pltpu.make_async_copy(rvn, acc_sh.at[bD.at[k - 1]], ssem).wait()
                elif gg == 1:
                    pltpu.make_async_copy(rvn, acc_sh.at[oD.at[GRP - 1]], ssem).wait()
                else:
                    @pl.when(g2 > 0)
                    def _():
                        pltpu.make_async_copy(rvn, acc_sh.at[oD.at[GRP - 1]], ssem).wait()
                if k == 0 and gg == 1:
                    @pl.when(g2 < 4)
                    def _():
                        _grp_load(g + 1, oS, oD)
                elif k == 0:
                    @pl.when(g2 > 0)
                    def _():
                        _grp_load(g + 1, oS, oD)
                if k < GRP - 1:
                    pltpu.async_copy(y_hbm.at[bS.at[k + 1]], rvn, gsem)
                elif gg == 0:
                    _grp_wait(g + 1, oS, oD)
                    pltpu.async_copy(y_hbm.at[oS.at[0]], rvn, gsem)
                else:
                    @pl.when(g2 < 4)
                    def _():
                        _grp_wait(g + 1, oS, oD)
                        pltpu.async_copy(y_hbm.at[oS.at[0]], rvn, gsem)
                pltpu.async_copy(rv, acc_sh.at[bD.at[k]], ssem, add=True)
        return 0

    lax.fori_loop(0, NGRP // 2, body, 0)

    pltpu.make_async_copy(rows1, acc_sh.at[dstgB.at[GRP - 1]], ssem).wait()

    plsc.subcore_barrier()
    pltpu.sync_copy(acc_sh.at[pl.ds(s * ROWS_PER_TILE, ROWS_PER_TILE)],
                    accp_hbm.at[c, pl.ds(s * ROWS_PER_TILE, ROWS_PER_TILE)])


@functools.partial(
    pl.kernel,
    out_type=jax.ShapeDtypeStruct((NC, N_PAD, D), jnp.float32),
    mesh=_mesh,
    scratch_types=(
        [pltpu.VMEM((GRP, CHUNK), jnp.int32)] * 4
        + [pltpu.VMEM((CHUNK, D), jnp.float32)] * 2
        + [pltpu.VMEM_SHARED((N_PAD, D), jnp.float32)]
        + [pltpu.SemaphoreType.DMA] * 3
    ),
    compiler_params=pltpu.CompilerParams(needs_layout_passes=False),
)
def _sc_scatter(y_hbm, src2_hbm, dst2_hbm, z2_hbm, accp_hbm,
                srcgA, srcgB, dstgA, dstgB,
                rows0, rows1, acc_sh, gsem, ssem, isem):
    _sc_scatter_body(y_hbm, src2_hbm, dst2_hbm, z2_hbm, accp_hbm,
                     srcgA, srcgB, dstgA, dstgB,
                     rows0, rows1, acc_sh, gsem, ssem, isem)


def _dinv_bcast(degp_ref):
    deg = 1.0 + jnp.sum(degp_ref[0], axis=0, keepdims=True)
    dinv = lax.rsqrt(deg)
    return lax.dot_general(dinv, jnp.ones((1, D), jnp.float32),
                           (((0,), (0,)), ((), ())),
                           preferred_element_type=jnp.float32)


def _tc_a_body(x_ref, w_ref, degp_ref, xw_ref, y_ref):
    db = _dinv_bcast(degp_ref)
    xw = jnp.dot(x_ref[...], w_ref[...], preferred_element_type=jnp.float32)
    xw_ref[...] = xw
    y_ref[...] = db * xw


def _tc_a(x, W1, degp):
    return pl.pallas_call(
        _tc_a_body,
        grid=(NBLK,),
        in_specs=[
            pl.BlockSpec((ROWS_BLK, D), lambda i: (i, 0)),
            pl.BlockSpec((D, D), lambda i: (0, 0)),
            pl.BlockSpec((1, NW, ROWS_BLK), lambda i: (i, 0, 0)),
        ],
        out_specs=[pl.BlockSpec((ROWS_BLK, D), lambda i: (i, 0))] * 2,
        out_shape=[jax.ShapeDtypeStruct((N_NODES, D), jnp.float32)] * 2,
    )(x, W1, degp)


def _tc_mid_body(accp_ref, degp_ref, xw1_ref, b_ref, a_ref, w_ref,
                 xw2_ref, y2_ref):
    acc = accp_ref[0] + accp_ref[1]
    db = _dinv_bcast(degp_ref)
    z = db * acc + db * db * xw1_ref[...] + b_ref[...]
    h = jnp.where(z >= 0, z, a_ref[...] * z)
    xw2 = jnp.dot(h, w_ref[...], preferred_element_type=jnp.float32)
    xw2_ref[...] = xw2
    y2_ref[...] = db * xw2


def _tc_mid(accp, degp, xw1, b1, a_b, W2):
    return pl.pallas_call(
        _tc_mid_body,
        grid=(NBLK,),
        in_specs=[
            pl.BlockSpec((NC, ROWS_BLK, D), lambda i: (0, i, 0)),
            pl.BlockSpec((1, NW, ROWS_BLK), lambda i: (i, 0, 0)),
            pl.BlockSpec((ROWS_BLK, D), lambda i: (i, 0)),
            pl.BlockSpec((1, D), lambda i: (0, 0)),
            pl.BlockSpec((1, D), lambda i: (0, 0)),
            pl.BlockSpec((D, D), lambda i: (0, 0)),
        ],
        out_specs=[pl.BlockSpec((ROWS_BLK, D), lambda i: (i, 0))] * 2,
        out_shape=[jax.ShapeDtypeStruct((N_NODES, D), jnp.float32)] * 2,
    )(accp, degp, xw1, b1, a_b, W2)


def _tc_out_body(accp_ref, degp_ref, xw2_ref, b_ref, a_ref, out_ref):
    acc = accp_ref[0] + accp_ref[1]
    db = _dinv_bcast(degp_ref)
    z = db * acc + db * db * xw2_ref[...] + b_ref[...]
    out_ref[...] = jnp.where(z >= 0, z, a_ref[...] * z)


def _tc_out(accp, degp, xw2, b2, a_b):
    return pl.pallas_call(
        _tc_out_body,
        grid=(NBLK,),
        in_specs=[
            pl.BlockSpec((NC, ROWS_BLK, D), lambda i: (0, i, 0)),
            pl.BlockSpec((1, NW, ROWS_BLK), lambda i: (i, 0, 0)),
            pl.BlockSpec((ROWS_BLK, D), lambda i: (i, 0)),
            pl.BlockSpec((1, D), lambda i: (0, 0)),
            pl.BlockSpec((1, D), lambda i: (0, 0)),
        ],
        out_specs=pl.BlockSpec((ROWS_BLK, D), lambda i: (i, 0)),
        out_shape=jax.ShapeDtypeStruct((N_NODES, D), jnp.float32),
    )(accp, degp, xw2, b2, a_b)


def kernel(x, edge_index, W1, b1, W2, b2, prelu_a):
    src = edge_index[0]
    dst = edge_index[1]
    b1r = jnp.reshape(b1, (1, D))
    b2r = jnp.reshape(b2, (1, D))
    a_b = jnp.broadcast_to(jnp.reshape(prelu_a, (1, 1)), (1, D))

    z1 = jnp.zeros((N_PAD,), jnp.float32)
    z2 = jnp.zeros((ROWS_PER_TILE, D), jnp.float32)
    n_pad_e = EROWS * CHUNK - N_EDGES
    pad_i = jnp.arange(n_pad_e, dtype=jnp.int32)
    src2 = jnp.concatenate([src, pad_i % N_NODES]).reshape(EROWS, CHUNK)
    dst2 = jnp.concatenate(
        [dst, N_NODES + pad_i % (N_PAD - N_NODES)]).reshape(EROWS, CHUNK)

    degp = _sc_deg(dst, z1)
    xw1, y1 = _tc_a(x, W1, degp)
    accp1 = _sc_scatter(y1, src2, dst2, z2)
    xw2, y2 = _tc_mid(accp1, degp, xw1, b1r, a_b, W2)
    accp2 = _sc_scatter(y2, src2, dst2, z2)
    return _tc_out(accp2, degp, xw2, b2r, a_b)


# speedup vs baseline: 1.2809x; 1.0148x over previous
import functools

import jax
import jax.numpy as jnp
from jax import lax
from jax.experimental import pallas as pl
from jax.experimental.pallas import tpu as pltpu
from jax.experimental.pallas import tpu_sc as plsc

N_NODES = 10000
D = 128
N_EDGES = 320000

NC, NS = 2, 16
NW = NC * NS
EPT = N_EDGES // NW
CHUNK = 128
NFULL = EPT // CHUNK
REM = EPT - NFULL * CHUNK
ROWS_BLK = 1024
NBLK = 10
N_PAD = ROWS_BLK * NBLK
ROWS_PER_TILE = N_PAD // NS
ZROWS = 128

_mesh = plsc.VectorSubcoreMesh(core_axis_name="c", subcore_axis_name="s")


def _sc_deg_body(dst_hbm, z1_hbm, degp_hbm, dst_v, deg_v):
    c = lax.axis_index("c")
    s = lax.axis_index("s")
    wid = c * NS + s
    one16 = jnp.ones((16,), jnp.float32)

    pltpu.sync_copy(z1_hbm, deg_v)
    pltpu.sync_copy(dst_hbm.at[pl.ds(wid * EPT, EPT)], dst_v)

    def scat_body(i, _):
        for u in range(5):
            idx = dst_v[pl.ds((i * 5 + u) * 16, 16)]
            plsc.addupdate_scatter(deg_v, [idx], one16)
        return 0

    lax.fori_loop(0, EPT // 80, scat_body, 0)
    for j in range(NBLK):
        pltpu.sync_copy(deg_v.at[pl.ds(j * ROWS_BLK, ROWS_BLK)],
                        degp_hbm.at[j, wid])


@functools.partial(
    pl.kernel,
    out_type=jax.ShapeDtypeStruct((NBLK, NW, ROWS_BLK), jnp.float32),
    mesh=_mesh,
    scratch_types=[
        pltpu.VMEM((EPT,), jnp.int32),
        pltpu.VMEM((N_PAD,), jnp.float32),
    ],
    compiler_params=pltpu.CompilerParams(needs_layout_passes=False),
)
def _sc_deg(dst_hbm, z1_hbm, degp_hbm, dst_v, deg_v):
    _sc_deg_body(dst_hbm, z1_hbm, degp_hbm, dst_v, deg_v)


def _sc_scatter_body(y_hbm, src_hbm, dst_hbm, z2_hbm, accp_hbm,
                     src_v0, src_v1, dst_v0, dst_v1, dst_v2, dst16_v,
                     rows0, rows1, acc_sh, gsem, ssem, isem):
    c = lax.axis_index("c")
    s = lax.axis_index("s")
    wid = c * NS + s
    base = wid * EPT

    pltpu.async_copy(z2_hbm, acc_sh.at[pl.ds(s * ROWS_PER_TILE, ROWS_PER_TILE)], ssem)
    pltpu.async_copy(src_hbm.at[pl.ds(base, CHUNK)], src_v0, isem)
    pltpu.async_copy(dst_hbm.at[pl.ds(base, CHUNK)], dst_v0, isem)
    pltpu.make_async_copy(src_hbm.at[pl.ds(base, CHUNK)], src_v0, isem).wait()
    pltpu.async_copy(y_hbm.at[src_v0], rows0, gsem)
    pltpu.make_async_copy(dst_hbm.at[pl.ds(base, CHUNK)], dst_v0, isem).wait()
    pltpu.make_async_copy(
        z2_hbm, acc_sh.at[pl.ds(s * ROWS_PER_TILE, ROWS_PER_TILE)], ssem).wait()
    plsc.subcore_barrier()
    src_ring = (src_v0, src_v1)
    dst_ring = (dst_v0, dst_v1, dst_v2)
    rows_ring = (rows0, rows1)
    UN = 6
    NJ = NFULL // UN

    pltpu.async_copy(src_hbm.at[pl.ds(base + CHUNK, CHUNK)], src_v1, isem)
    pltpu.async_copy(dst_hbm.at[pl.ds(base + CHUNK, CHUNK)], dst_v1, isem)

    def body(j, _):
        for k in range(UN):
            sv, svn = src_ring[k % 2], src_ring[(k + 1) % 2]
            dv = dst_ring[k % 3]
            rv, rvn = rows_ring[k % 2], rows_ring[(k + 1) % 2]
            svp, dvp = src_ring[(k + 1) % 2], dst_ring[(k + 1) % 3]
            off = base + (j * UN + k) * CHUNK

            if k == 5:
                @pl.when(j < NJ - 1)
                def _():
                    pltpu.make_async_copy(src_hbm.at[pl.ds(off + CHUNK, CHUNK)], svp, isem).wait()
                    pltpu.make_async_copy(dst_hbm.at[pl.ds(off + CHUNK, CHUNK)], dvp, isem).wait()
            else:
                pltpu.make_async_copy(src_hbm.at[pl.ds(off + CHUNK, CHUNK)], svp, isem).wait()
                pltpu.make_async_copy(dst_hbm.at[pl.ds(off + CHUNK, CHUNK)], dvp, isem).wait()

            pltpu.make_async_copy(y_hbm.at[sv], rv, gsem).wait()

            pv, pd = rows_ring[(k + 1) % 2], dst_ring[(k + 2) % 3]
            if k == 0:
                @pl.when(j > 0)
                def _():
                    pltpu.make_async_copy(pv, acc_sh.at[pd], ssem).wait()
            else:
                pltpu.make_async_copy(pv, acc_sh.at[pd], ssem).wait()

            if k == 5:
                @pl.when(j < NJ - 1)
                def _():
                    pltpu.async_copy(y_hbm.at[svn], rvn, gsem)
            else:
                pltpu.async_copy(y_hbm.at[svn], rvn, gsem)

            pltpu.async_copy(rv, acc_sh.at[dv], ssem, add=True)

            if k >= 4:
                @pl.when(j < NJ - 1)
                def _():
                    pltpu.async_copy(src_hbm.at[pl.ds(off + 2 * CHUNK, CHUNK)], sv, isem)
                    pltpu.async_copy(dst_hbm.at[pl.ds(off + 2 * CHUNK, CHUNK)], dst_ring[(k + 2) % 3], isem)
            else:
                pltpu.async_copy(src_hbm.at[pl.ds(off + 2 * CHUNK, CHUNK)], sv, isem)
                pltpu.async_copy(dst_hbm.at[pl.ds(off + 2 * CHUNK, CHUNK)], dst_ring[(k + 2) % 3], isem)
        return 0

    lax.fori_loop(0, NJ, body, 0)

    pltpu.make_async_copy(rows1, acc_sh.at[dst_v2], ssem).wait()

    off = base + NFULL * CHUNK
    pltpu.sync_copy(src_hbm.at[pl.ds(off, REM)], src_v0.at[pl.ds(0, REM)])
    pltpu.async_copy(y_hbm.at[src_v0.at[pl.ds(0, REM)]],
                     rows0.at[pl.ds(0, REM)], gsem).wait()
    pltpu.sync_copy(dst_hbm.at[pl.ds(off, REM)], dst16_v)
    pltpu.sync_copy(rows0.at[pl.ds(0, REM)], acc_sh.at[dst16_v], add=True)

    plsc.subcore_barrier()
    pltpu.sync_copy(acc_sh.at[pl.ds(s * ROWS_PER_TILE, ROWS_PER_TILE)],
                    accp_hbm.at[c, pl.ds(s * ROWS_PER_TILE, ROWS_PER_TILE)])


@functools.partial(
    pl.kernel,
    out_type=jax.ShapeDtypeStruct((NC, N_PAD, D), jnp.float32),
    mesh=_mesh,
    scratch_types=(
        [pltpu.VMEM((CHUNK,), jnp.int32)] * 5
        + [pltpu.VMEM((REM,), jnp.int32)]
        + [pltpu.VMEM((CHUNK, D), jnp.float32)] * 2
        + [pltpu.VMEM_SHARED((N_PAD, D), jnp.float32)]
        + [pltpu.SemaphoreType.DMA] * 3
    ),
    compiler_params=pltpu.CompilerParams(needs_layout_passes=False),
)
def _sc_scatter(y_hbm, src_hbm, dst_hbm, z2_hbm, accp_hbm,
                src_v0, src_v1, dst_v0, dst_v1, dst_v2, dst16_v,
                rows0, rows1, acc_sh, gsem, ssem, isem):
    _sc_scatter_body(y_hbm, src_hbm, dst_hbm, z2_hbm, accp_hbm,
                     src_v0, src_v1, dst_v0, dst_v1, dst_v2, dst16_v,
                     rows0, rows1, acc_sh, gsem, ssem, isem)


def _dinv_bcast(degp_ref):
    deg = 1.0 + jnp.sum(degp_ref[0], axis=0, keepdims=True)
    dinv = lax.rsqrt(deg)
    return lax.dot_general(dinv, jnp.ones((1, D), jnp.float32),
                           (((0,), (0,)), ((), ())),
                           preferred_element_type=jnp.float32)


def _tc_a_body(x_ref, w_ref, degp_ref, xw_ref, y_ref):
    db = _dinv_bcast(degp_ref)
    xw = jnp.dot(x_ref[...], w_ref[...], preferred_element_type=jnp.float32)
    xw_ref[...] = xw
    y_ref[...] = db * xw


def _tc_a(x, W1, degp):
    return pl.pallas_call(
        _tc_a_body,
        grid=(NBLK,),
        in_specs=[
            pl.BlockSpec((ROWS_BLK, D), lambda i: (i, 0)),
            pl.BlockSpec((D, D), lambda i: (0, 0)),
            pl.BlockSpec((1, NW, ROWS_BLK), lambda i: (i, 0, 0)),
        ],
        out_specs=[pl.BlockSpec((ROWS_BLK, D), lambda i: (i, 0))] * 2,
        out_shape=[jax.ShapeDtypeStruct((N_NODES, D), jnp.float32)] * 2,
    )(x, W1, degp)


def _tc_mid_body(accp_ref, degp_ref, xw1_ref, b_ref, a_ref, w_ref,
                 xw2_ref, y2_ref):
    acc = accp_ref[0] + accp_ref[1]
    db = _dinv_bcast(degp_ref)
    z = db * acc + db * db * xw1_ref[...] + b_ref[...]
    h = jnp.where(z >= 0, z, a_ref[...] * z)
    xw2 = jnp.dot(h, w_ref[...], preferred_element_type=jnp.float32)
    xw2_ref[...] = xw2
    y2_ref[...] = db * xw2


def _tc_mid(accp, degp, xw1, b1, a_b, W2):
    return pl.pallas_call(
        _tc_mid_body,
        grid=(NBLK,),
        in_specs=[
            pl.BlockSpec((NC, ROWS_BLK, D), lambda i: (0, i, 0)),
            pl.BlockSpec((1, NW, ROWS_BLK), lambda i: (i, 0, 0)),
            pl.BlockSpec((ROWS_BLK, D), lambda i: (i, 0)),
            pl.BlockSpec((1, D), lambda i: (0, 0)),
            pl.BlockSpec((1, D), lambda i: (0, 0)),
            pl.BlockSpec((D, D), lambda i: (0, 0)),
        ],
        out_specs=[pl.BlockSpec((ROWS_BLK, D), lambda i: (i, 0))] * 2,
        out_shape=[jax.ShapeDtypeStruct((N_NODES, D), jnp.float32)] * 2,
    )(accp, degp, xw1, b1, a_b, W2)


def _tc_out_body(accp_ref, degp_ref, xw2_ref, b_ref, a_ref, out_ref):
    acc = accp_ref[0] + accp_ref[1]
    db = _dinv_bcast(degp_ref)
    z = db * acc + db * db * xw2_ref[...] + b_ref[...]
    out_ref[...] = jnp.where(z >= 0, z, a_ref[...] * z)


def _tc_out(accp, degp, xw2, b2, a_b):
    return pl.pallas_call(
        _tc_out_body,
        grid=(NBLK,),
        in_specs=[
            pl.BlockSpec((NC, ROWS_BLK, D), lambda i: (0, i, 0)),
            pl.BlockSpec((1, NW, ROWS_BLK), lambda i: (i, 0, 0)),
            pl.BlockSpec((ROWS_BLK, D), lambda i: (i, 0)),
            pl.BlockSpec((1, D), lambda i: (0, 0)),
            pl.BlockSpec((1, D), lambda i: (0, 0)),
        ],
        out_specs=pl.BlockSpec((ROWS_BLK, D), lambda i: (i, 0)),
        out_shape=jax.ShapeDtypeStruct((N_NODES, D), jnp.float32),
    )(accp, degp, xw2, b2, a_b)


def kernel(x, edge_index, W1, b1, W2, b2, prelu_a):
    src = edge_index[0]
    dst = edge_index[1]
    b1r = jnp.reshape(b1, (1, D))
    b2r = jnp.reshape(b2, (1, D))
    a_b = jnp.broadcast_to(jnp.reshape(prelu_a, (1, 1)), (1, D))

    z1 = jnp.zeros((N_PAD,), jnp.float32)
    z2 = jnp.zeros((ROWS_PER_TILE, D), jnp.float32)

    degp = _sc_deg(dst, z1)
    xw1, y1 = _tc_a(x, W1, degp)
    accp1 = _sc_scatter(y1, src, dst, z2)
    xw2, y2 = _tc_mid(accp1, degp, xw1, b1r, a_b, W2)
    accp2 = _sc_scatter(y2, src, dst, z2)
    return _tc_out(accp2, degp, xw2, b2r, a_b)
